# Initial kernel scaffold; baseline (speedup 1.0000x reference)
import jax
import jax.numpy as jnp
from jax.experimental import pallas as pl


def kernel(boxes, scores):
    raise NotImplementedError("write your pallas kernel here")


# speedup vs baseline: 11.5307x; 11.5307x over previous
import jax
import jax.numpy as jnp
from jax import lax
from jax.experimental import pallas as pl

_SCORE_T = 0.001
_IOU_T = 0.7
_MAX_DET = 300
_PRE_NMS = 4096
_BLK = 256


def _nms_keep_body(rx1, ry1, rx2, ry2, cx1, cy1, cx2, cy2, valid, keep_out):
    n = _PRE_NMS
    m = _BLK
    nb = n // m
    col = lax.broadcasted_iota(jnp.int32, (1, n), 1)
    li = lax.broadcasted_iota(jnp.int32, (m, m), 0)
    lj = lax.broadcasted_iota(jnp.int32, (m, m), 1)
    tri = (li < lj).astype(jnp.float32)

    x1c = cx1[...]
    y1c = cy1[...]
    x2c = cx2[...]
    y2c = cy2[...]
    area_c = jnp.maximum(x2c - x1c, 0.0) * jnp.maximum(y2c - y1c, 0.0)

    keep = valid[...]

    for b in range(nb):
        r0 = b * m
        x1r = rx1[pl.ds(r0, m), :]
        y1r = ry1[pl.ds(r0, m), :]
        x2r = rx2[pl.ds(r0, m), :]
        y2r = ry2[pl.ds(r0, m), :]
        area_r = jnp.maximum(x2r - x1r, 0.0) * jnp.maximum(y2r - y1r, 0.0)
        ltx = jnp.maximum(x1r, x1c)
        lty = jnp.maximum(y1r, y1c)
        rbx = jnp.minimum(x2r, x2c)
        rby = jnp.minimum(y2r, y2c)
        w = jnp.maximum(rbx - ltx, 0.0)
        h = jnp.maximum(rby - lty, 0.0)
        inter = w * h
        union = area_r + area_c - inter
        iou = inter / jnp.maximum(union, 1e-9)
        sup_f = (iou > _IOU_T).astype(jnp.float32)

        sbb = sup_f[:, r0:r0 + m] * tri
        kb0 = keep[:, r0:r0 + m]

        def w_cond(c):
            return c[1]

        def w_body(c, kb0=kb0, sbb=sbb):
            kb, _ = c
            s = lax.dot_general(kb, sbb, (((1,), (0,)), ((), ())),
                                preferred_element_type=jnp.float32)
            kb_new = jnp.where(s > 0.0, 0.0, kb0)
            return kb_new, jnp.any(kb_new != kb)

        kb, _ = lax.while_loop(w_cond, w_body, (kb0, True))

        sup_later = lax.dot_general(kb, sup_f, (((1,), (0,)), ((), ())),
                                    preferred_element_type=jnp.float32)
        pieces = []
        if r0 > 0:
            pieces.append(keep[:, :r0])
        pieces.append(kb)
        if r0 + m < n:
            pieces.append(keep[:, r0 + m:])
        keep = jnp.concatenate(pieces, axis=1) if len(pieces) > 1 else kb
        keep = jnp.where((col >= r0 + m) & (sup_later > 0.0), 0.0, keep)

    keep_out[...] = keep


def _nms_one(bx, sc):
    n_cls = sc.shape[-1]
    flat = sc.reshape(-1)
    flat = jnp.where(flat >= _SCORE_T, flat, -1.0)
    top_s, top_i = lax.top_k(flat, _PRE_NMS)
    box_idx = top_i // n_cls
    labels = top_i % n_cls
    cand = bx[box_idx]
    max_c = jnp.max(bx) + 1.0
    off = labels.astype(bx.dtype)[:, None] * max_c
    shifted = cand + off
    valid_f = (top_s > 0.0).astype(jnp.float32)[None, :]

    rows = [shifted[:, i:i + 1] for i in range(4)]
    cols = [shifted[:, i][None, :] for i in range(4)]

    keep_f = pl.pallas_call(
        _nms_keep_body,
        out_shape=jax.ShapeDtypeStruct((1, _PRE_NMS), jnp.float32),
    )(*rows, *cols, valid_f)

    keep = keep_f[0] > 0.5
    kept_scores = jnp.where(keep, top_s, -1.0)
    fs, fi = lax.top_k(kept_scores, _MAX_DET)
    sel_ok = fs > 0.0
    out_boxes = jnp.where(sel_ok[:, None], cand[fi], 0.0)
    out_scores = jnp.where(sel_ok, fs, 0.0)
    out_labels = jnp.where(sel_ok, labels[fi], 0).astype(jnp.int32)
    n_valid = jnp.sum(sel_ok).astype(jnp.int32)
    return out_boxes, out_scores, out_labels, n_valid


def kernel(boxes, scores):
    bdim = boxes.shape[0]
    outs = [_nms_one(boxes[i], scores[i]) for i in range(bdim)]
    return tuple(jnp.stack([o[k] for o in outs]) for k in range(4))
